# baseline (speedup 1.0000x reference)
import jax
import jax.numpy as jnp
import numpy as np
from jax import lax
from jax.experimental import pallas as pl
from jax.experimental.pallas import tpu as pltpu
from jax.experimental.pallas import tpu_sc as plsc

N = 10000
D = 128
N_CLS = 64
E = 320000

NC = 2
NS = 16
NW = NC * NS
L = 16

CHUNK = 128
WE = E // NW
NCHF = WE // CHUNK
TAIL = WE - NCHF * CHUNK
RPW = 640
N_PAD = NS * RPW

_MESH = plsc.VectorSubcoreMesh(core_axis_name="c", subcore_axis_name="s")


def _sc_agg_body(p_hbm, srcc, dstc, agg_out,
                 idx_s, idx_d, rows, idx_st, idx_dt, rows_t, accum,
                 sis0, sis1, sid0, sid1, sg0, sg1):
    c = lax.axis_index("c")
    s = lax.axis_index("s")
    w = c * NS + s
    zeros16 = jnp.zeros((L,), jnp.float32)
    zbuf = rows.at[0]

    @pl.loop(0, CHUNK)
    def _(i):
        for j in range(D // L):
            zbuf[i, pl.ds(j * L, L)] = zeros16

    for k in range(RPW // CHUNK):
        pltpu.async_copy(zbuf, accum.at[pl.ds(s * RPW + k * CHUNK, CHUNK)], sg1)
    for k in range(RPW // CHUNK):
        pltpu.make_async_copy(zbuf, accum.at[pl.ds(s * RPW + k * CHUNK, CHUNK)],
                              sg1).wait()

    plsc.subcore_barrier()

    e0 = w * WE

    pltpu.async_copy(srcc.at[pl.ds(e0, CHUNK)], idx_s.at[0], sis0)
    pltpu.async_copy(dstc.at[pl.ds(e0, CHUNK)], idx_d.at[0], sid0)
    pltpu.make_async_copy(srcc.at[pl.ds(e0, CHUNK)], idx_s.at[0], sis0).wait()
    pltpu.async_copy(p_hbm.at[idx_s.at[0]], rows.at[0], sg0)
    pltpu.async_copy(srcc.at[pl.ds(e0 + CHUNK, CHUNK)], idx_s.at[1], sis1)
    pltpu.async_copy(dstc.at[pl.ds(e0 + CHUNK, CHUNK)], idx_d.at[1], sid1)

    @pl.loop(0, NCHF, step=2)
    def _(g):
        pltpu.make_async_copy(srcc.at[pl.ds(e0 + CHUNK, CHUNK)], idx_s.at[1],
                              sis1).wait()
        pltpu.async_copy(p_hbm.at[idx_s.at[1]], rows.at[1], sg1)
        pltpu.make_async_copy(p_hbm.at[idx_s.at[0]], rows.at[0], sg0).wait()
        pltpu.make_async_copy(dstc.at[pl.ds(e0, CHUNK)], idx_d.at[0],
                              sid0).wait()
        pltpu.sync_copy(rows.at[0], accum.at[idx_d.at[0]], add=True)

        @pl.when(g + 2 < NCHF)
        def _():
            o = e0 + (g + 2) * CHUNK
            pltpu.async_copy(srcc.at[pl.ds(o, CHUNK)], idx_s.at[0], sis0)
            pltpu.async_copy(dstc.at[pl.ds(o, CHUNK)], idx_d.at[0], sid0)

        @pl.when(g + 2 < NCHF)
        def _():
            pltpu.make_async_copy(srcc.at[pl.ds(e0, CHUNK)], idx_s.at[0],
                                  sis0).wait()
            pltpu.async_copy(p_hbm.at[idx_s.at[0]], rows.at[0], sg0)

        pltpu.make_async_copy(p_hbm.at[idx_s.at[1]], rows.at[1], sg1).wait()
        pltpu.make_async_copy(dstc.at[pl.ds(e0, CHUNK)], idx_d.at[1],
                              sid1).wait()
        pltpu.sync_copy(rows.at[1], accum.at[idx_d.at[1]], add=True)

        @pl.when(g + 3 < NCHF)
        def _():
            o = e0 + (g + 3) * CHUNK
            pltpu.async_copy(srcc.at[pl.ds(o, CHUNK)], idx_s.at[1], sis1)
            pltpu.async_copy(dstc.at[pl.ds(o, CHUNK)], idx_d.at[1], sid1)

    to = e0 + NCHF * CHUNK
    pltpu.sync_copy(srcc.at[pl.ds(to, TAIL)], idx_st.at[0])
    pltpu.sync_copy(dstc.at[pl.ds(to, TAIL)], idx_dt.at[0])
    pltpu.sync_copy(p_hbm.at[idx_st.at[0]], rows_t)
    pltpu.sync_copy(rows_t, accum.at[idx_dt.at[0]], add=True)

    plsc.subcore_barrier()

    r0 = s * RPW
    o0 = c * N_PAD + s * RPW
    pltpu.async_copy(accum.at[pl.ds(r0, CHUNK)], rows.at[0], sg0)
    for k in range(RPW // CHUNK):
        b = k % 2
        nxt = 1 - b
        if k + 1 < RPW // CHUNK:
            pltpu.async_copy(accum.at[pl.ds(r0 + (k + 1) * CHUNK, CHUNK)],
                             rows.at[nxt], sg1 if nxt else sg0)
        pltpu.make_async_copy(accum.at[pl.ds(r0 + k * CHUNK, CHUNK)],
                              rows.at[b], sg1 if b else sg0).wait()
        pltpu.sync_copy(rows.at[b], agg_out.at[pl.ds(o0 + k * CHUNK, CHUNK)])


_sc_agg = pl.kernel(
    _sc_agg_body,
    out_type=jax.ShapeDtypeStruct((NC * N_PAD, D), jnp.float32),
    mesh=_MESH,
    scratch_types=[
        pltpu.VMEM((2, CHUNK), jnp.int32),
        pltpu.VMEM((2, CHUNK), jnp.int32),
        pltpu.VMEM((2, CHUNK, D), jnp.float32),
        pltpu.VMEM((1, TAIL), jnp.int32),
        pltpu.VMEM((1, TAIL), jnp.int32),
        pltpu.VMEM((TAIL, D), jnp.float32),
        pltpu.VMEM_SHARED((N_PAD, D), jnp.float32),
        pltpu.SemaphoreType.DMA,
        pltpu.SemaphoreType.DMA,
        pltpu.SemaphoreType.DMA,
        pltpu.SemaphoreType.DMA,
        pltpu.SemaphoreType.DMA,
        pltpu.SemaphoreType.DMA,
    ],
)

EB = E // 10
SUB = 2000
NHI = N_PAD // D


def _proj_kernel(x_ref, wn_ref, ws_ref, b_ref, d_ref, p_ref, s_ref, deg_ref):
    x = x_ref[...]
    p_ref[...] = jnp.dot(x, wn_ref[...], preferred_element_type=jnp.float32)
    s_ref[...] = (
        jnp.dot(x, ws_ref[...], preferred_element_type=jnp.float32) + b_ref[...]
    )

    @pl.when(pl.program_id(0) == 0)
    def _():
        deg_ref[...] = jnp.zeros((D, NHI), jnp.float32)

    acc = jnp.zeros((D, NHI), jnp.float32)
    for b in range(EB // SUB):
        db = d_ref[0, 0, pl.ds(b * SUB, SUB)]
        hi = jnp.right_shift(db, 7)
        lo = jnp.bitwise_and(db, 127)
        oh_hi = (lax.broadcasted_iota(jnp.int32, (NHI, SUB), 0)
                 == hi[None, :]).astype(jnp.bfloat16)
        oh_lo_t = (lax.broadcasted_iota(jnp.int32, (D, SUB), 0)
                   == lo[None, :]).astype(jnp.bfloat16)
        acc = acc + lax.dot_general(
            oh_lo_t, oh_hi, (((1,), (1,)), ((), ())),
            preferred_element_type=jnp.float32)
    deg_ref[...] += acc


def _mid_kernel(s_ref, a_ref, d_ref, wn_ref, ws_ref, b_ref, p_ref, s2_ref):
    deg = jnp.maximum(d_ref[0, 0], 1.0)[:, None]
    agg = (a_ref[0] + a_ref[1]) / deg
    h = jnp.maximum(s_ref[...] + agg, 0.0)
    p_ref[...] = jnp.dot(h, wn_ref[...], preferred_element_type=jnp.float32)
    s2_ref[...] = (
        jnp.dot(h, ws_ref[...], preferred_element_type=jnp.float32) + b_ref[...]
    )


def _head_kernel(s_ref, a_ref, d_ref, wfc_ref, b_ref, o_ref):
    deg = jnp.maximum(d_ref[0, 0], 1.0)[:, None]
    agg = (a_ref[0] + a_ref[1]) / deg
    h = jnp.maximum(s_ref[...] + agg, 0.0)
    o_ref[...] = (
        jnp.dot(h, wfc_ref[...], preferred_element_type=jnp.float32) + b_ref[...]
    )


_BR = 1024
_GRID = -(-N // _BR)


def _row_spec(d):
    return pl.BlockSpec((_BR, d), lambda i: (i, 0))


def _part_spec(d):
    return pl.BlockSpec((2, _BR, d), lambda i: (0, i, 0))


_DEG_SPEC = pl.BlockSpec((1, 1, _BR), lambda i: (i, 0, 0))


def _full_spec(a, b):
    return pl.BlockSpec((a, b), lambda i: (0, 0))


def kernel(x, edge_index, W_self1, W_neigh1, b1, W_self2, W_neigh2, b2, W_fc, b_fc):
    src = edge_index[0].astype(jnp.int32)
    dst = edge_index[1].astype(jnp.int32)
    b1r = b1.reshape(1, D)
    b2r = b2.reshape(1, D)
    bfr = b_fc.reshape(1, N_CLS)

    dst_e = dst.reshape(10, 1, EB)
    proj = pl.pallas_call(
        _proj_kernel,
        grid=(_GRID,),
        in_specs=[_row_spec(D), _full_spec(D, D), _full_spec(D, D),
                  _full_spec(1, D), pl.BlockSpec((1, 1, EB), lambda i: (i, 0, 0))],
        out_specs=[_row_spec(D), _row_spec(D),
                   pl.BlockSpec((D, NHI), lambda i: (0, 0))],
        out_shape=[jax.ShapeDtypeStruct((N, D), jnp.float32),
                   jax.ShapeDtypeStruct((N, D), jnp.float32),
                   jax.ShapeDtypeStruct((D, NHI), jnp.float32)],
    )
    p1, s1, deg80 = proj(x, W_neigh1, W_self1, b1r, dst_e)

    deg2 = deg80.T.reshape(N_PAD // _BR, 1, _BR)
    agg1 = _sc_agg(p1, src, dst).reshape(NC, N_PAD, D)

    mid = pl.pallas_call(
        _mid_kernel,
        grid=(_GRID,),
        in_specs=[_row_spec(D), _part_spec(D), _DEG_SPEC,
                  _full_spec(D, D), _full_spec(D, D), _full_spec(1, D)],
        out_specs=[_row_spec(D), _row_spec(D)],
        out_shape=[jax.ShapeDtypeStruct((N, D), jnp.float32)] * 2,
    )
    p2, s2 = mid(s1, agg1, deg2, W_neigh2, W_self2, b2r)

    agg2 = _sc_agg(p2, src, dst).reshape(NC, N_PAD, D)

    head = pl.pallas_call(
        _head_kernel,
        grid=(_GRID,),
        in_specs=[_row_spec(D), _part_spec(D), _DEG_SPEC,
                  _full_spec(D, N_CLS), _full_spec(1, N_CLS)],
        out_specs=_row_spec(N_CLS),
        out_shape=jax.ShapeDtypeStruct((N, N_CLS), jnp.float32),
    )
    return head(s2, agg2, deg2, W_fc, bfr)


# speedup vs baseline: 1.0419x; 1.0419x over previous
import jax
import jax.numpy as jnp
import numpy as np
from jax import lax
from jax.experimental import pallas as pl
from jax.experimental.pallas import tpu as pltpu
from jax.experimental.pallas import tpu_sc as plsc

N = 10000
D = 128
N_CLS = 64
E = 320000

NC = 2
NS = 16
NW = NC * NS
L = 16

CHUNK = 128
WE = E // NW
NCHF = WE // CHUNK
TAIL = WE - NCHF * CHUNK
RPW = 640
N_PAD = NS * RPW

_MESH = plsc.VectorSubcoreMesh(core_axis_name="c", subcore_axis_name="s")


def _sc_agg_body(p_hbm, srcc, dstc, agg_out,
                 idx_s, idx_d, rows, idx_st, idx_dt, rows_t, accum,
                 sis0, sis1, sid0, sid1, sg0, sg1):
    c = lax.axis_index("c")
    s = lax.axis_index("s")
    w = c * NS + s
    zeros16 = jnp.zeros((L,), jnp.float32)
    zbuf = rows.at[0]

    @pl.loop(0, CHUNK)
    def _(i):
        for j in range(D // L):
            zbuf[i, pl.ds(j * L, L)] = zeros16

    for k in range(RPW // CHUNK):
        pltpu.async_copy(zbuf, accum.at[pl.ds(s * RPW + k * CHUNK, CHUNK)], sg1)
    for k in range(RPW // CHUNK):
        pltpu.make_async_copy(zbuf, accum.at[pl.ds(s * RPW + k * CHUNK, CHUNK)],
                              sg1).wait()

    plsc.subcore_barrier()

    e0 = w * WE

    pltpu.async_copy(srcc.at[pl.ds(e0, CHUNK)], idx_s.at[0], sis0)
    pltpu.async_copy(dstc.at[pl.ds(e0, CHUNK)], idx_d.at[0], sid0)
    pltpu.make_async_copy(srcc.at[pl.ds(e0, CHUNK)], idx_s.at[0], sis0).wait()
    pltpu.async_copy(p_hbm.at[idx_s.at[0]], rows.at[0], sg0)
    pltpu.async_copy(srcc.at[pl.ds(e0 + CHUNK, CHUNK)], idx_s.at[1], sis1)
    pltpu.async_copy(dstc.at[pl.ds(e0 + CHUNK, CHUNK)], idx_d.at[1], sid1)

    @pl.loop(0, NCHF, step=2)
    def _(g):
        pltpu.make_async_copy(srcc.at[pl.ds(e0 + CHUNK, CHUNK)],
                              idx_s.at[1], sis1).wait()
        pltpu.async_copy(p_hbm.at[idx_s.at[1]], rows.at[1], sg1)
        pltpu.make_async_copy(p_hbm.at[idx_s.at[0]], rows.at[0], sg0).wait()
        pltpu.make_async_copy(dstc.at[pl.ds(e0, CHUNK)], idx_d.at[0],
                              sid0).wait()
        pltpu.sync_copy(rows.at[0], accum.at[idx_d.at[0]], add=True)

        @pl.when(g + 2 < NCHF)
        def _():
            o = e0 + (g + 2) * CHUNK
            pltpu.async_copy(srcc.at[pl.ds(o, CHUNK)], idx_s.at[0], sis0)
            pltpu.async_copy(dstc.at[pl.ds(o, CHUNK)], idx_d.at[0], sid0)

        @pl.when(g + 2 < NCHF)
        def _():
            pltpu.make_async_copy(srcc.at[pl.ds(e0, CHUNK)], idx_s.at[0],
                                  sis0).wait()
            pltpu.async_copy(p_hbm.at[idx_s.at[0]], rows.at[0], sg0)

        pltpu.make_async_copy(p_hbm.at[idx_s.at[1]], rows.at[1], sg1).wait()
        pltpu.make_async_copy(dstc.at[pl.ds(e0, CHUNK)], idx_d.at[1],
                              sid1).wait()
        pltpu.sync_copy(rows.at[1], accum.at[idx_d.at[1]], add=True)

        @pl.when(g + 3 < NCHF)
        def _():
            o = e0 + (g + 3) * CHUNK
            pltpu.async_copy(srcc.at[pl.ds(o, CHUNK)], idx_s.at[1], sis1)
            pltpu.async_copy(dstc.at[pl.ds(o, CHUNK)], idx_d.at[1], sid1)

    to = e0 + NCHF * CHUNK
    pltpu.sync_copy(srcc.at[pl.ds(to, TAIL)], idx_st.at[0])
    pltpu.sync_copy(dstc.at[pl.ds(to, TAIL)], idx_dt.at[0])
    pltpu.sync_copy(p_hbm.at[idx_st.at[0]], rows_t)
    pltpu.sync_copy(rows_t, accum.at[idx_dt.at[0]], add=True)

    plsc.subcore_barrier()

    r0 = s * RPW
    o0 = c * N_PAD + s * RPW
    pltpu.async_copy(accum.at[pl.ds(r0, CHUNK)], rows.at[0], sg0)
    for k in range(RPW // CHUNK):
        b = k % 2
        nxt = 1 - b
        if k + 1 < RPW // CHUNK:
            pltpu.async_copy(accum.at[pl.ds(r0 + (k + 1) * CHUNK, CHUNK)],
                             rows.at[nxt], sg1 if nxt else sg0)
        pltpu.make_async_copy(accum.at[pl.ds(r0 + k * CHUNK, CHUNK)],
                              rows.at[b], sg1 if b else sg0).wait()
        pltpu.sync_copy(rows.at[b], agg_out.at[pl.ds(o0 + k * CHUNK, CHUNK)])


_sc_agg = pl.kernel(
    _sc_agg_body,
    out_type=jax.ShapeDtypeStruct((NC * N_PAD, D), jnp.float32),
    mesh=_MESH,
    scratch_types=[
        pltpu.VMEM((2, CHUNK), jnp.int32),
        pltpu.VMEM((2, CHUNK), jnp.int32),
        pltpu.VMEM((2, CHUNK, D), jnp.float32),
        pltpu.VMEM((1, TAIL), jnp.int32),
        pltpu.VMEM((1, TAIL), jnp.int32),
        pltpu.VMEM((TAIL, D), jnp.float32),
        pltpu.VMEM_SHARED((N_PAD, D), jnp.float32),
        pltpu.SemaphoreType.DMA,
        pltpu.SemaphoreType.DMA,
        pltpu.SemaphoreType.DMA,
        pltpu.SemaphoreType.DMA,
        pltpu.SemaphoreType.DMA,
        pltpu.SemaphoreType.DMA,
    ],
)

EB = E // 10
SUB = 6400
NHI = N_PAD // D


def _proj_kernel(x_ref, wn_ref, ws_ref, b_ref, d_ref, p_ref, s_ref, deg_ref):
    x = x_ref[...]
    p_ref[...] = jnp.dot(x, wn_ref[...], preferred_element_type=jnp.float32)
    s_ref[...] = (
        jnp.dot(x, ws_ref[...], preferred_element_type=jnp.float32) + b_ref[...]
    )

    @pl.when(pl.program_id(0) == 0)
    def _():
        acc = jnp.zeros((D, NHI), jnp.float32)
        for b in range(E // SUB):
            db = d_ref[pl.ds(b * SUB, SUB)]
            hi = jnp.right_shift(db, 7)
            lo = jnp.bitwise_and(db, 127)
            oh_hi = (lax.broadcasted_iota(jnp.int32, (NHI, SUB), 0)
                     == hi[None, :]).astype(jnp.bfloat16)
            oh_lo_t = (lax.broadcasted_iota(jnp.int32, (D, SUB), 0)
                       == lo[None, :]).astype(jnp.bfloat16)
            acc = acc + lax.dot_general(
                oh_lo_t, oh_hi, (((1,), (1,)), ((), ())),
                preferred_element_type=jnp.float32)
        deg_ref[...] = acc


def _mid_kernel(s_ref, a_ref, d_ref, wn_ref, ws_ref, b_ref, p_ref, s2_ref):
    deg = jnp.maximum(d_ref[0, 0], 1.0)[:, None]
    agg = (a_ref[0] + a_ref[1]) / deg
    h = jnp.maximum(s_ref[...] + agg, 0.0)
    p_ref[...] = jnp.dot(h, wn_ref[...], preferred_element_type=jnp.float32)
    s2_ref[...] = (
        jnp.dot(h, ws_ref[...], preferred_element_type=jnp.float32) + b_ref[...]
    )


def _head_kernel(s_ref, a_ref, d_ref, wfc_ref, b_ref, o_ref):
    deg = jnp.maximum(d_ref[0, 0], 1.0)[:, None]
    agg = (a_ref[0] + a_ref[1]) / deg
    h = jnp.maximum(s_ref[...] + agg, 0.0)
    o_ref[...] = (
        jnp.dot(h, wfc_ref[...], preferred_element_type=jnp.float32) + b_ref[...]
    )


_BR = 1024
_GRID = -(-N // _BR)


def _row_spec(d):
    return pl.BlockSpec((_BR, d), lambda i: (i, 0))


def _part_spec(d):
    return pl.BlockSpec((2, _BR, d), lambda i: (0, i, 0))


_DEG_SPEC = pl.BlockSpec((1, 1, _BR), lambda i: (i, 0, 0))


def _full_spec(a, b):
    return pl.BlockSpec((a, b), lambda i: (0, 0))


def kernel(x, edge_index, W_self1, W_neigh1, b1, W_self2, W_neigh2, b2, W_fc, b_fc):
    src = edge_index[0]
    dst = edge_index[1]
    b1r = b1.reshape(1, D)
    b2r = b2.reshape(1, D)
    bfr = b_fc.reshape(1, N_CLS)

    proj = pl.pallas_call(
        _proj_kernel,
        grid=(_GRID,),
        in_specs=[_row_spec(D), _full_spec(D, D), _full_spec(D, D),
                  _full_spec(1, D), pl.BlockSpec((E,), lambda i: (0,))],
        out_specs=[_row_spec(D), _row_spec(D),
                   pl.BlockSpec((D, NHI), lambda i: (0, 0))],
        out_shape=[jax.ShapeDtypeStruct((N, D), jnp.float32),
                   jax.ShapeDtypeStruct((N, D), jnp.float32),
                   jax.ShapeDtypeStruct((D, NHI), jnp.float32)],
    )
    p1, s1, deg80 = proj(x, W_neigh1, W_self1, b1r, dst)

    deg2 = deg80.T.reshape(N_PAD // _BR, 1, _BR)
    agg1 = _sc_agg(p1, src, dst).reshape(NC, N_PAD, D)

    mid = pl.pallas_call(
        _mid_kernel,
        grid=(_GRID,),
        in_specs=[_row_spec(D), _part_spec(D), _DEG_SPEC,
                  _full_spec(D, D), _full_spec(D, D), _full_spec(1, D)],
        out_specs=[_row_spec(D), _row_spec(D)],
        out_shape=[jax.ShapeDtypeStruct((N, D), jnp.float32)] * 2,
    )
    p2, s2 = mid(s1, agg1, deg2, W_neigh2, W_self2, b2r)

    agg2 = _sc_agg(p2, src, dst).reshape(NC, N_PAD, D)

    head = pl.pallas_call(
        _head_kernel,
        grid=(_GRID,),
        in_specs=[_row_spec(D), _part_spec(D), _DEG_SPEC,
                  _full_spec(D, N_CLS), _full_spec(1, N_CLS)],
        out_specs=_row_spec(N_CLS),
        out_shape=jax.ShapeDtypeStruct((N, N_CLS), jnp.float32),
    )
    return head(s2, agg2, deg2, W_fc, bfr)
